# baseline (speedup 1.0000x reference)
import numpy as np
import jax
import jax.numpy as jnp
from jax.experimental import pallas as pl

_EPS = 1e-06


def _alloc_const(n):
    u = np.full((n,), _EPS, dtype=np.float32)
    u_prod = np.cumprod(u, dtype=np.float32)
    one_minus = (np.float32(1.0) - u).astype(np.float32)
    scores = np.concatenate([one_minus[:1], one_minus[1:] * u_prod[:-1]])
    return scores.astype(np.float32)


def _body(mem_ref, key_ref, beta_ref, erase_ref, wv_ref, ag_ref, wg_ref,
          fg_ref, prd_ref, alloc_ref, out_ref):
    for _bi in range(mem_ref.shape[0]):
        _sub_body(_bi, mem_ref, key_ref, beta_ref, erase_ref, wv_ref, ag_ref,
                  wg_ref, fg_ref, prd_ref, alloc_ref, out_ref)


def _sub_body(bi, mem_ref, key_ref, beta_ref, erase_ref, wv_ref, ag_ref,
              wg_ref, fg_ref, prd_ref, alloc_ref, out_ref):
    b = pl.program_id(0) * mem_ref.shape[0] + bi
    mem = mem_ref[bi]
    N, W = mem.shape
    beta = beta_ref[b, 0]
    ag = ag_ref[b, 0]
    wg = wg_ref[b, 0]

    keyrow = key_ref[b][None, :]
    onesrow = jnp.ones((1, W), dtype=mem.dtype)
    dot2 = jax.lax.dot_general(keyrow, mem, (((1,), (1,)), ((), ())),
                               preferred_element_type=jnp.float32)
    sq = mem * mem
    msq2 = jax.lax.dot_general(onesrow, sq, (((1,), (1,)), ((), ())),
                               preferred_element_type=jnp.float32)

    knorm = jnp.sqrt(jnp.sum(key_ref[b] * key_ref[b]))
    norm = knorm * jnp.sqrt(msq2)
    score = beta * (dot2 / (norm + _EPS))
    e = jnp.exp(score)
    inv_se = 1.0 / jnp.sum(e)

    prd = prd_ref[bi]
    R = prd.shape[0]
    phi = 1.0 - fg_ref[b, 0] * prd[0:1]
    for r in range(1, R):
        phi = phi * (1.0 - fg_ref[b, r] * prd[r:r + 1])

    alloc = alloc_ref[0][None, :]
    wd = (wg * ag) * alloc + (wg * (1.0 - ag) * inv_se) * e

    erase = erase_ref[b][None, :]
    wv = wv_ref[b][None, :]
    zerosrow = jnp.zeros((1, W), dtype=mem.dtype)
    lhs = jnp.concatenate([phi, -wd * phi, wd], axis=0)
    rhs = jnp.concatenate(
        [jnp.concatenate([onesrow, zerosrow], axis=1),
         jnp.concatenate([erase, zerosrow], axis=1),
         jnp.concatenate([zerosrow, wv], axis=1)], axis=0)
    m = jax.lax.dot_general(lhs, rhs, (((0,), (0,)), ((), ())),
                            preferred_element_type=jnp.float32)
    out_ref[bi] = mem * m[:, :W] + m[:, W:]


def kernel(memory, write_content_key, write_beta, erase_vector, write_vector,
           alloc_gate, write_gate, free_gates, prev_read_dist):
    B, N, W = memory.shape
    R = free_gates.shape[1]
    alloc = jnp.asarray(_alloc_const(N))[None, :]

    full = lambda a: pl.BlockSpec(a.shape, lambda b: (0,) * a.ndim)
    return pl.pallas_call(
        _body,
        grid=(B // 2,),
        in_specs=[
            pl.BlockSpec((2, N, W), lambda b: (b, 0, 0)),
            full(write_content_key),
            full(write_beta),
            full(erase_vector),
            full(write_vector),
            full(alloc_gate),
            full(write_gate),
            full(free_gates),
            pl.BlockSpec((2, R, N), lambda b: (b, 0, 0)),
            full(alloc),
        ],
        out_specs=pl.BlockSpec((2, N, W), lambda b: (b, 0, 0)),
        out_shape=jax.ShapeDtypeStruct((B, N, W), memory.dtype),
    )(memory, write_content_key, write_beta, erase_vector, write_vector,
      alloc_gate, write_gate, free_gates, prev_read_dist, alloc)
